# Initial kernel scaffold; baseline (speedup 1.0000x reference)
import jax
import jax.numpy as jnp
from jax.experimental import pallas as pl


def kernel(points, point_indices, memory_bank):
    raise NotImplementedError("write your pallas kernel here")


# speedup vs baseline: 27.1198x; 27.1198x over previous
import functools

import jax
import jax.numpy as jnp
from jax.experimental import pallas as pl

_T = 0.07
_K = 4096
_N_BANK = 100000
_D = 64
_N_PTS = 1024
_ROWS_PER_BLOCK = 16
_BISECT_ITERS = 26


def _loss_kernel(pts_ref, bankT_ref, idx_ref, sim_ref, term_ref):
    pts = pts_ref[...]
    norm = jnp.sqrt(jnp.sum(pts * pts, axis=1, keepdims=True))
    ptsn = pts / norm
    sims = jnp.dot(ptsn, bankT_ref[...], preferred_element_type=jnp.float32)
    sim_ref[...] = sims

    r = sims.shape[0]
    kf = jnp.float32(_K)

    lo0 = jnp.full((r, 1), -1.5, jnp.float32)
    hi0 = jnp.full((r, 1), 1.5, jnp.float32)

    def body(_, carry):
        lo, hi = carry
        mid = 0.5 * (lo + hi)
        cnt = jnp.sum(jnp.where(sims > mid, 1.0, 0.0), axis=1, keepdims=True)
        ge = cnt >= kf
        return jnp.where(ge, mid, lo), jnp.where(ge, hi, mid)

    lo, hi = jax.lax.fori_loop(0, _BISECT_ITERS, body, (lo0, hi0))

    inv_t = jnp.float32(1.0 / _T)
    mask = sims > hi
    cnt_hi = jnp.sum(jnp.where(mask, 1.0, 0.0), axis=1, keepdims=True)
    sum_gt = jnp.sum(jnp.where(mask, jnp.exp(sims * inv_t), 0.0), axis=1,
                     keepdims=True)
    topk_sum = sum_gt + (kf - cnt_hi) * jnp.exp(hi * inv_t)

    idx = idx_ref[0]
    cols = jax.lax.broadcasted_iota(jnp.int32, (r, _N_BANK), 1)
    hit = cols == idx.reshape(r, 1)
    pos = jnp.sum(jnp.where(hit, sims, 0.0), axis=1, keepdims=True)
    pos_exp = jnp.exp(pos * inv_t)

    term = jnp.log(pos_exp / topk_sum + jnp.float32(1e-7))
    term_ref[0] = term.reshape(1, r)


def _run(points, point_indices, memory_bank, interpret=False):
    nb = _N_PTS // _ROWS_PER_BLOCK
    r = _ROWS_PER_BLOCK
    bank_t = memory_bank.T
    idx3 = point_indices.astype(jnp.int32).reshape(nb, 1, r)

    sims, terms = pl.pallas_call(
        _loss_kernel,
        grid=(nb,),
        in_specs=[
            pl.BlockSpec((r, _D), lambda i: (i, 0)),
            pl.BlockSpec((_D, _N_BANK), lambda i: (0, 0)),
            pl.BlockSpec((1, 1, r), lambda i: (i, 0, 0)),
        ],
        out_specs=[
            pl.BlockSpec((r, _N_BANK), lambda i: (i, 0)),
            pl.BlockSpec((1, 1, r), lambda i: (i, 0, 0)),
        ],
        out_shape=[
            jax.ShapeDtypeStruct((_N_PTS, _N_BANK), jnp.float32),
            jax.ShapeDtypeStruct((nb, 1, r), jnp.float32),
        ],
        interpret=interpret,
    )(points, bank_t, idx3)

    loss = -jnp.mean(terms)
    return (loss, sims)


def kernel(points, point_indices, memory_bank):
    return _run(points, point_indices, memory_bank)


# speedup vs baseline: 34.3807x; 1.2677x over previous
import functools

import jax
import jax.numpy as jnp
from jax.experimental import pallas as pl

_T = 0.07
_K = 4096
_N_BANK = 100000
_D = 64
_N_PTS = 1024
_ROWS_PER_BLOCK = 16
_BISECT_ITERS = 18


def _loss_kernel(pts_ref, bankT_ref, idx_ref, sim_ref, term_ref):
    pts = pts_ref[...]
    norm = jnp.sqrt(jnp.sum(pts * pts, axis=1, keepdims=True))
    ptsn = pts / norm
    sims = jnp.dot(ptsn, bankT_ref[...], preferred_element_type=jnp.float32)
    sim_ref[...] = sims

    r = sims.shape[0]
    kf = jnp.float32(_K)

    lo0 = jnp.full((r, 1), -1.5, jnp.float32)
    hi0 = jnp.full((r, 1), 1.5, jnp.float32)

    def body(_, carry):
        lo, hi = carry
        mid = 0.5 * (lo + hi)
        cnt = jnp.sum(sims > mid, axis=1, keepdims=True)
        ge = cnt >= _K
        return jnp.where(ge, mid, lo), jnp.where(ge, hi, mid)

    lo, hi = jax.lax.fori_loop(0, _BISECT_ITERS, body, (lo0, hi0))

    inv_t = jnp.float32(1.0 / _T)
    mask = sims > hi
    cnt_hi = jnp.sum(mask, axis=1, keepdims=True).astype(jnp.float32)
    sum_gt = jnp.sum(jnp.where(mask, jnp.exp(sims * inv_t), 0.0), axis=1,
                     keepdims=True)
    topk_sum = sum_gt + (kf - cnt_hi) * jnp.exp(hi * inv_t)

    idx = idx_ref[0]
    cols = jax.lax.broadcasted_iota(jnp.int32, (r, _N_BANK), 1)
    hit = cols == idx.reshape(r, 1)
    pos = jnp.sum(jnp.where(hit, sims, 0.0), axis=1, keepdims=True)
    pos_exp = jnp.exp(pos * inv_t)

    term = jnp.log(pos_exp / topk_sum + jnp.float32(1e-7))
    term_ref[0] = term.reshape(1, r)


def _run(points, point_indices, memory_bank, interpret=False):
    nb = _N_PTS // _ROWS_PER_BLOCK
    r = _ROWS_PER_BLOCK
    bank_t = memory_bank.T
    idx3 = point_indices.astype(jnp.int32).reshape(nb, 1, r)

    sims, terms = pl.pallas_call(
        _loss_kernel,
        grid=(nb,),
        in_specs=[
            pl.BlockSpec((r, _D), lambda i: (i, 0)),
            pl.BlockSpec((_D, _N_BANK), lambda i: (0, 0)),
            pl.BlockSpec((1, 1, r), lambda i: (i, 0, 0)),
        ],
        out_specs=[
            pl.BlockSpec((r, _N_BANK), lambda i: (i, 0)),
            pl.BlockSpec((1, 1, r), lambda i: (i, 0, 0)),
        ],
        out_shape=[
            jax.ShapeDtypeStruct((_N_PTS, _N_BANK), jnp.float32),
            jax.ShapeDtypeStruct((nb, 1, r), jnp.float32),
        ],
        interpret=interpret,
    )(points, bank_t, idx3)

    loss = -jnp.mean(terms)
    return (loss, sims)


def kernel(points, point_indices, memory_bank):
    return _run(points, point_indices, memory_bank)


# speedup vs baseline: 44.9807x; 1.3083x over previous
import functools

import jax
import jax.numpy as jnp
from jax.experimental import pallas as pl

_T = 0.07
_K = 4096
_N_BANK = 100000
_D = 64
_N_PTS = 1024
_ROWS_PER_BLOCK = 32
_BISECT_ITERS = 12


def _loss_kernel(pts_ref, bankT_ref, idx_ref, sim_ref, term_ref):
    pts = pts_ref[...]
    norm = jnp.sqrt(jnp.sum(pts * pts, axis=1, keepdims=True))
    ptsn = (pts / norm).astype(jnp.bfloat16)
    sims = jnp.dot(ptsn, bankT_ref[...], preferred_element_type=jnp.float32)
    sim_ref[...] = sims

    r = sims.shape[0]
    kf = jnp.float32(_K)

    lo0 = jnp.full((r, 1), -1.5, jnp.float32)
    hi0 = jnp.full((r, 1), 1.5, jnp.float32)

    def body(_, carry):
        lo, hi = carry
        mid = 0.5 * (lo + hi)
        cnt = jnp.sum(sims > mid, axis=1, keepdims=True)
        ge = cnt >= _K
        return jnp.where(ge, mid, lo), jnp.where(ge, hi, mid)

    lo, hi = jax.lax.fori_loop(0, _BISECT_ITERS, body, (lo0, hi0))

    inv_t = jnp.float32(1.0 / _T)
    mask = sims > hi
    cnt_hi = jnp.sum(mask, axis=1, keepdims=True).astype(jnp.float32)
    sum_gt = jnp.sum(jnp.where(mask, jnp.exp(sims * inv_t), 0.0), axis=1,
                     keepdims=True)
    topk_sum = sum_gt + (kf - cnt_hi) * jnp.exp(0.5 * (lo + hi) * inv_t)

    idx = idx_ref[0]
    cols = jax.lax.broadcasted_iota(jnp.int32, (r, _N_BANK), 1)
    hit = cols == idx.reshape(r, 1)
    pos = jnp.sum(jnp.where(hit, sims, 0.0), axis=1, keepdims=True)
    pos_exp = jnp.exp(pos * inv_t)

    term = jnp.log(pos_exp / topk_sum + jnp.float32(1e-7))
    term_ref[0] = term.reshape(1, r)


def _run(points, point_indices, memory_bank, interpret=False):
    nb = _N_PTS // _ROWS_PER_BLOCK
    r = _ROWS_PER_BLOCK
    bank_t = memory_bank.T.astype(jnp.bfloat16)
    idx3 = point_indices.astype(jnp.int32).reshape(nb, 1, r)

    sims, terms = pl.pallas_call(
        _loss_kernel,
        grid=(nb,),
        in_specs=[
            pl.BlockSpec((r, _D), lambda i: (i, 0)),
            pl.BlockSpec((_D, _N_BANK), lambda i: (0, 0)),
            pl.BlockSpec((1, 1, r), lambda i: (i, 0, 0)),
        ],
        out_specs=[
            pl.BlockSpec((r, _N_BANK), lambda i: (i, 0)),
            pl.BlockSpec((1, 1, r), lambda i: (i, 0, 0)),
        ],
        out_shape=[
            jax.ShapeDtypeStruct((_N_PTS, _N_BANK), jnp.float32),
            jax.ShapeDtypeStruct((nb, 1, r), jnp.float32),
        ],
        interpret=interpret,
    )(points, bank_t, idx3)

    loss = -jnp.mean(terms)
    return (loss, sims)


def kernel(points, point_indices, memory_bank):
    return _run(points, point_indices, memory_bank)
